# baseline (speedup 1.0000x reference)
import functools
import jax
import jax.numpy as jnp
from jax import lax
from jax.experimental import pallas as pl
from jax.experimental.pallas import tpu as pltpu
from jax.experimental.pallas import tpu_sc as plsc

H1 = 8
C1 = 8
F1 = H1 * C1
C2 = 16
K = 128
NTILES = 32
RB = 128

_PREC = jax.lax.Precision.HIGHEST


def _cdiv(a, b):
    return (a + b - 1) // b


def _blocks(rows):
    out = [RB] * (rows // RB)
    if rows % RB:
        out.append(rows % RB)
    return out


def _tc1_body(x_ref, w_ref, t_out):
    t_out[...] = jnp.dot(x_ref[...], w_ref[...],
                         preferred_element_type=jnp.float32, precision=_PREC)


def _tc1(x_p, W1ext, npad, d):
    bn = npad // 8
    return pl.pallas_call(
        _tc1_body,
        grid=(npad // bn,),
        in_specs=[pl.BlockSpec((bn, d), lambda i: (i, 0)),
                  pl.BlockSpec((d, 128), lambda i: (0, 0))],
        out_specs=pl.BlockSpec((bn, 128), lambda i: (i, 0)),
        out_shape=jax.ShapeDtypeStruct((npad, 128), jnp.float32),
    )(x_p, W1ext)


def _tc2_body(acc_ref, w2_ref, p1_ref, r8_ref, m2_ref, b1_ref, t_out):
    a = acc_ref[0] + acc_ref[1]
    den = jnp.dot(a, p1_ref[...], preferred_element_type=jnp.float32,
                  precision=_PREC)
    dr = jnp.dot(1.0 / (den + 1e-16), r8_ref[...],
                 preferred_element_type=jnp.float32, precision=_PREC)
    z = a[:, :F1] * dr + b1_ref[...]
    e = jnp.where(z > 0, z, jnp.exp(jnp.minimum(z, 0.0)) - 1.0)
    h2 = jnp.dot(e, w2_ref[...], preferred_element_type=jnp.float32,
                 precision=_PREC)
    t_out[...] = jnp.dot(h2, m2_ref[...], preferred_element_type=jnp.float32,
                         precision=_PREC)


def _tc2(acc1, W2, P1, R8, M2, b1r, npad):
    bn = npad // 8
    return pl.pallas_call(
        _tc2_body,
        grid=(npad // bn,),
        in_specs=[pl.BlockSpec((2, bn, 128), lambda i: (0, i, 0)),
                  pl.BlockSpec((F1, C2), lambda i: (0, 0)),
                  pl.BlockSpec((128, H1), lambda i: (0, 0)),
                  pl.BlockSpec((H1, F1), lambda i: (0, 0)),
                  pl.BlockSpec((C2, 128), lambda i: (0, 0)),
                  pl.BlockSpec((1, F1), lambda i: (0, 0))],
        out_specs=pl.BlockSpec((bn, 128), lambda i: (i, 0)),
        out_shape=jax.ShapeDtypeStruct((npad, 128), jnp.float32),
    )(acc1, W2, P1, R8, M2, b1r)


def _tc3_body(acc_ref, p2_ref, b2_ref, out_ref):
    a = acc_ref[0] + acc_ref[1]
    den = jnp.dot(a, p2_ref[...], preferred_element_type=jnp.float32,
                  precision=_PREC)
    o = a[:, :C2] / (den + 1e-16) + b2_ref[...]
    m = jnp.max(o, axis=1, keepdims=True)
    t = o - m
    lse = jnp.log(jnp.sum(jnp.exp(t), axis=1, keepdims=True))
    out_ref[...] = t - lse


def _tc3(acc2, P2, b2r, npad):
    bn = npad // 8
    return pl.pallas_call(
        _tc3_body,
        grid=(npad // bn,),
        in_specs=[pl.BlockSpec((2, bn, 128), lambda i: (0, i, 0)),
                  pl.BlockSpec((128, C2), lambda i: (0, 0)),
                  pl.BlockSpec((1, C2), lambda i: (0, 0))],
        out_specs=pl.BlockSpec((bn, C2), lambda i: (i, 0)),
        out_shape=jax.ShapeDtypeStruct((npad, C2), jnp.float32),
    )(acc2, P2, b2r)


def _sc1_body(nrb, rows_t, ch_t,
              src_hbm, dst_hbm, t1_hbm, out_hbm,
              acc_sp, sidx, didx, sbuf, dbuf, msg):
    c = lax.axis_index("c")
    s = lax.axis_index("s")
    row0 = s * rows_t
    zeros16 = jnp.zeros((16,), jnp.float32)

    def zrow(r, carry):
        for q in range(8):
            msg[r, pl.ds(16 * q, 16)] = zeros16
        return carry
    lax.fori_loop(0, RB, zrow, 0)
    off = 0
    for nr in _blocks(rows_t):
        pltpu.sync_copy(msg.at[pl.ds(0, nr)], acc_sp.at[pl.ds(row0 + off, nr)])
        off += nr
    plsc.subcore_barrier()

    iot = lax.iota(jnp.int32, 16)
    colb = lax.shift_right_logical(iot, 3)
    ior = lax.bitwise_or(iot, 8)

    def chunk(jc, carry):
        pltpu.sync_copy(src_hbm.at[c, s, jc], sidx)
        pltpu.sync_copy(dst_hbm.at[c, s, jc], didx)
        pltpu.sync_copy(t1_hbm.at[sidx], sbuf)
        pltpu.sync_copy(t1_hbm.at[didx], dbuf)

        def edge(k, ecarry):
            srow = sbuf[k, pl.ds(F1, 16)]
            drow = dbuf[k, pl.ds(F1, 16)]
            e = srow + drow.at[ior].get(mode="promise_in_bounds")
            w = jnp.exp(jnp.maximum(e, 0.2 * e))
            msg[k, pl.ds(F1, 16)] = w
            for q in range(4):
                wx = w.at[colb + 2 * q].get(mode="promise_in_bounds")
                msg[k, pl.ds(16 * q, 16)] = sbuf[k, pl.ds(16 * q, 16)] * wx
            return ecarry
        lax.fori_loop(0, K, edge, 0)
        pltpu.sync_copy(msg, acc_sp.at[didx], add=True)
        return carry
    lax.fori_loop(0, ch_t, chunk, 0)

    plsc.subcore_barrier()
    off = 0
    for nr in _blocks(rows_t):
        r = row0 + off
        pltpu.sync_copy(acc_sp.at[pl.ds(r, nr)], msg.at[pl.ds(0, nr)])
        pltpu.sync_copy(msg.at[pl.ds(0, nr)], out_hbm.at[c, pl.ds(r, nr)])
        off += nr


def _sc2_body(nrb, rows_t, ch_t,
              src_hbm, dst_hbm, t2_hbm, out_hbm,
              acc_sp, sidx, didx, sbuf, dbuf, msg):
    c = lax.axis_index("c")
    s = lax.axis_index("s")
    row0 = s * rows_t
    zeros16 = jnp.zeros((16,), jnp.float32)

    def zrow(r, carry):
        for q in range(8):
            msg[r, pl.ds(16 * q, 16)] = zeros16
        return carry
    lax.fori_loop(0, RB, zrow, 0)
    off = 0
    for nr in _blocks(rows_t):
        pltpu.sync_copy(msg.at[pl.ds(0, nr)], acc_sp.at[pl.ds(row0 + off, nr)])
        off += nr
    plsc.subcore_barrier()

    def chunk(jc, carry):
        pltpu.sync_copy(src_hbm.at[c, s, jc], sidx)
        pltpu.sync_copy(dst_hbm.at[c, s, jc], didx)
        pltpu.sync_copy(t2_hbm.at[sidx], sbuf)
        pltpu.sync_copy(t2_hbm.at[didx], dbuf)

        def edge(k, ecarry):
            e = sbuf[k, pl.ds(16, 16)] + dbuf[k, pl.ds(32, 16)]
            w = jnp.exp(jnp.maximum(e, 0.2 * e))
            msg[k, pl.ds(16, 16)] = w
            msg[k, pl.ds(0, 16)] = sbuf[k, pl.ds(0, 16)] * w
            return ecarry
        lax.fori_loop(0, K, edge, 0)
        pltpu.sync_copy(msg, acc_sp.at[didx], add=True)
        return carry
    lax.fori_loop(0, ch_t, chunk, 0)

    plsc.subcore_barrier()
    off = 0
    for nr in _blocks(rows_t):
        r = row0 + off
        pltpu.sync_copy(acc_sp.at[pl.ds(r, nr)], msg.at[pl.ds(0, nr)])
        pltpu.sync_copy(msg.at[pl.ds(0, nr)], out_hbm.at[c, pl.ds(r, nr)])
        off += nr


def _sc_mesh():
    return plsc.VectorSubcoreMesh(core_axis_name="c", subcore_axis_name="s",
                                  num_cores=2, num_subcores=16)


def _sc1(src_r, dst_r, t1, npad, ch_t):
    nrb = npad // 16 // RB
    rows_t = npad // 16
    body = functools.partial(_sc1_body, nrb, rows_t, ch_t)
    f = pl.kernel(
        body,
        out_type=jax.ShapeDtypeStruct((2, npad, 128), jnp.float32),
        mesh=_sc_mesh(),
        scratch_types=[
            pltpu.VMEM_SHARED((npad, 128), jnp.float32),
            pltpu.VMEM((K,), jnp.int32),
            pltpu.VMEM((K,), jnp.int32),
            pltpu.VMEM((K, 128), jnp.float32),
            pltpu.VMEM((K, 128), jnp.float32),
            pltpu.VMEM((K, 128), jnp.float32),
        ],
    )
    return f(src_r, dst_r, t1)


def _sc2(src_r, dst_r, t2, npad, ch_t):
    nrb = npad // 16 // RB
    rows_t = npad // 16
    body = functools.partial(_sc2_body, nrb, rows_t, ch_t)
    f = pl.kernel(
        body,
        out_type=jax.ShapeDtypeStruct((2, npad, 128), jnp.float32),
        mesh=_sc_mesh(),
        scratch_types=[
            pltpu.VMEM_SHARED((npad, 128), jnp.float32),
            pltpu.VMEM((K,), jnp.int32),
            pltpu.VMEM((K,), jnp.int32),
            pltpu.VMEM((K, 128), jnp.float32),
            pltpu.VMEM((K, 128), jnp.float32),
            pltpu.VMEM((K, 128), jnp.float32),
        ],
    )
    return f(src_r, dst_r, t2)


def kernel(x, edge_index, W1, a_src1, a_dst1, b1, W2, a_src2, a_dst2, b2):
    n, d = x.shape
    e_in = edge_index.shape[1]
    etot = e_in + n
    ch_t = _cdiv(etot, K * NTILES)
    epad = ch_t * K * NTILES
    npad = _cdiv(n + 1, 1264) * 1264

    loop = jnp.arange(n, dtype=jnp.int32)
    src = jnp.concatenate([edge_index[0].astype(jnp.int32), loop,
                           jnp.full((epad - etot,), n, jnp.int32)])
    dst = jnp.concatenate([edge_index[1].astype(jnp.int32), loop,
                           jnp.full((epad - etot,), n, jnp.int32)])
    src_r = src.reshape(2, 16, ch_t, K)
    dst_r = dst.reshape(2, 16, ch_t, K)
    x_p = jnp.pad(x, ((0, npad - n), (0, 0)))

    i64 = jnp.arange(F1)
    Asd = (jnp.zeros((F1, 16), jnp.float32)
           .at[i64, i64 // C1].set(a_src1.reshape(F1))
           .at[i64, 8 + i64 // C1].set(a_dst1.reshape(F1)))
    W1ext = jnp.concatenate(
        [W1, jnp.dot(W1, Asd, precision=_PREC),
         jnp.zeros((d, 48), jnp.float32)], axis=1)
    P1 = jnp.zeros((128, H1), jnp.float32).at[F1 + jnp.arange(H1), jnp.arange(H1)].set(1.0)
    R8 = jnp.zeros((H1, F1), jnp.float32).at[i64 // C1, i64].set(1.0)
    i16 = jnp.arange(C2)
    M2 = (jnp.zeros((C2, 128), jnp.float32)
          .at[i16, i16].set(1.0)
          .at[:, 16:32].set(jnp.broadcast_to(a_src2.reshape(C2, 1), (C2, 16)))
          .at[:, 32:48].set(jnp.broadcast_to(a_dst2.reshape(C2, 1), (C2, 16))))
    P2 = jnp.zeros((128, C2), jnp.float32).at[C2 + i16, i16].set(1.0)

    t1 = _tc1(x_p, W1ext, npad, d)
    acc1 = _sc1(src_r, dst_r, t1, npad, ch_t)
    t2 = _tc2(acc1, W2, P1, R8, M2, b1.reshape(1, F1), npad)
    acc2 = _sc2(src_r, dst_r, t2, npad, ch_t)
    out = _tc3(acc2, P2, b2.reshape(1, C2), npad)
    return out[:n]
